# Initial kernel scaffold; baseline (speedup 1.0000x reference)
import jax
import jax.numpy as jnp
from jax.experimental import pallas as pl


def kernel(attn_bias, spatial_pos, x, spatial_pos_encoder_weight):
    raise NotImplementedError("write your pallas kernel here")


# speedup vs baseline: 6.5451x; 6.5451x over previous
import functools

import jax
import jax.numpy as jnp
from jax import lax
from jax.experimental import pallas as pl
from jax.experimental.pallas import tpu as pltpu
from jax.experimental.pallas import tpu_sc as plsc

G = 8
N = 512
H = 16
V = 512
NP = N + 1
NC = 2
NS = 16
NW = NC * NS
WPG = NW // G
ROWS_W = N // WPG
R = 4
NCHUNK = ROWS_W // R
L = 16


def _sc_body(bias_hbm, idx_hbm, w_hbm, out_hbm,
             table, bias_v, idxb0, idxb1, ob0, ob1, brow,
             semi0, semi1, semo0, semo1):
    wid = lax.axis_index("s") * NC + lax.axis_index("c")
    g = wid // WPG
    slot = wid % WPG
    row0 = slot * ROWS_W

    pltpu.sync_copy(w_hbm, table)
    pltpu.sync_copy(bias_hbm, bias_v)
    vb2 = bias_v[...] * 2.0

    def _fold(i, carry):
        table[i, :] = table[i, :] + vb2
        return carry
    lax.fori_loop(0, V, _fold, 0)

    iota = lax.iota(jnp.int32, L)
    hsplats = [jnp.zeros((L,), jnp.int32) + h for h in range(H)]
    rsplats = [jnp.zeros((L,), jnp.int32) + r for r in range(R)]

    @pl.when(slot == 0)
    def _():
        def _fill(i, carry):
            brow[pl.ds(i * L, L)] = vb2
            return carry
        lax.fori_loop(0, 33, _fill, 0)
        for h in range(H):
            pltpu.sync_copy(brow.at[pl.ds(0, NP)], out_hbm.at[g, h, 0, :])

    pltpu.async_copy(idx_hbm.at[g, pl.ds(row0, R), :], idxb0, semi0)

    def do_chunk(c, idxb, ob, semi, semo, idxb_next, semi_next):
        @pl.when(c + 1 < NCHUNK)
        def _():
            pltpu.async_copy(
                idx_hbm.at[g, pl.ds(row0 + (c + 1) * R, R), :],
                idxb_next, semi_next)

        pltpu.make_async_copy(
            idx_hbm.at[g, pl.ds(row0, R), :], idxb, semi).wait()

        @pl.when(c >= 2)
        def _():
            for h in range(H):
                pltpu.make_async_copy(
                    ob.at[h], out_hbm.at[g, h, pl.ds(1, R), :], semo).wait()

        for r in range(R):
            plsc.store_scatter(
                ob, [iota, rsplats[r], jnp.zeros((L,), jnp.int32)], vb2)

        for r in range(R):
            def _grp(t, carry, r=r):
                j0 = t * L
                idx16 = idxb[r, pl.ds(j0, L)]
                posc = iota + (j0 + 1)
                for h in range(H):
                    vals = plsc.load_gather(table, [idx16, hsplats[h]])
                    plsc.store_scatter(
                        ob, [hsplats[h], rsplats[r], posc], vals)
                return carry
            lax.fori_loop(0, N // L, _grp, 0)

        prow = row0 + c * R + 1
        for h in range(H):
            pltpu.async_copy(
                ob.at[h], out_hbm.at[g, h, pl.ds(prow, R), :], semo)

    def loop_body(k, carry):
        do_chunk(2 * k, idxb0, ob0, semi0, semo0, idxb1, semi1)
        do_chunk(2 * k + 1, idxb1, ob1, semi1, semo1, idxb0, semi0)
        return carry
    lax.fori_loop(0, NCHUNK // 2, loop_body, 0)

    for ob, semo in ((ob0, semo0), (ob1, semo1)):
        for h in range(H):
            pltpu.make_async_copy(
                ob.at[h], out_hbm.at[g, h, pl.ds(1, R), :], semo).wait()


@jax.jit
def _graph_attn_bias(attn_bias, spatial_pos, weight):
    mesh = plsc.VectorSubcoreMesh(core_axis_name="c", subcore_axis_name="s")
    run = pl.kernel(
        _sc_body,
        out_type=jax.ShapeDtypeStruct((G, H, NP, NP), jnp.float32),
        mesh=mesh,
        compiler_params=pltpu.CompilerParams(
            use_tc_tiling_on_sc=False, needs_layout_passes=False),
        scratch_types=[
            pltpu.VMEM((V, H), jnp.float32),
            pltpu.VMEM((L,), jnp.float32),
            pltpu.VMEM((R, N), jnp.int32),
            pltpu.VMEM((R, N), jnp.int32),
            pltpu.VMEM((H, R, NP), jnp.float32),
            pltpu.VMEM((H, R, NP), jnp.float32),
            pltpu.VMEM((544,), jnp.float32),
            pltpu.SemaphoreType.DMA,
            pltpu.SemaphoreType.DMA,
            pltpu.SemaphoreType.DMA,
            pltpu.SemaphoreType.DMA,
        ],
    )
    return run(attn_bias, spatial_pos, weight)


def kernel(attn_bias, spatial_pos, x, spatial_pos_encoder_weight):
    del x
    idx = spatial_pos.astype(jnp.int32)
    bias16 = jnp.broadcast_to(attn_bias.reshape(1), (L,))
    return _graph_attn_bias(bias16, idx, spatial_pos_encoder_weight)
